# initial kernel scaffold (unmeasured)
import jax
import jax.numpy as jnp
from jax import lax
from jax.experimental import pallas as pl
from jax.experimental.pallas import tpu as pltpu

N_DEV = 32


def kernel(x, w_mat):
    m_per, k = x.shape
    n = w_mat.shape[1]
    m_tot = N_DEV * m_per

    def body(x_ref, w_ref, out_ref, gx_ref, amax_ref,
             ring_send_sem, ring_recv_sems, sc_send_sem, sc_recv_sem):
        my = lax.axis_index("i")
        left = jnp.mod(my - 1, N_DEV)
        right = jnp.mod(my + 1, N_DEV)

        barrier_sem = pltpu.get_barrier_semaphore()
        pl.semaphore_signal(barrier_sem, inc=1, device_id=(left,),
                            device_id_type=pl.DeviceIdType.MESH)
        pl.semaphore_signal(barrier_sem, inc=1, device_id=(right,),
                            device_id_type=pl.DeviceIdType.MESH)
        pl.semaphore_wait(barrier_sem, 2)

        w = w_ref[...]

        gx_ref[pl.ds(my, 1)] = x_ref[...].reshape(1, m_per, k)
        y0 = jnp.maximum(
            jnp.dot(x_ref[...], w, preferred_element_type=jnp.float32), 0.0)
        out_ref[pl.ds(my * m_per, m_per), :] = y0
        amax = jnp.max(y0)

        for h in range(N_DEV - 1):
            send_origin = jnp.mod(my - h, N_DEV)
            recv_origin = jnp.mod(my - h - 1, N_DEV)
            send = pltpu.make_async_remote_copy(
                src_ref=gx_ref.at[send_origin],
                dst_ref=gx_ref.at[send_origin],
                send_sem=ring_send_sem,
                recv_sem=ring_recv_sems.at[send_origin],
                device_id=(right,),
                device_id_type=pl.DeviceIdType.MESH,
            )
            send.start()
            send.wait_send()
            recv = pltpu.make_async_remote_copy(
                src_ref=gx_ref.at[recv_origin],
                dst_ref=gx_ref.at[recv_origin],
                send_sem=ring_send_sem,
                recv_sem=ring_recv_sems.at[recv_origin],
                device_id=(right,),
                device_id_type=pl.DeviceIdType.MESH,
            )
            recv.wait_recv()
            xb = gx_ref[pl.ds(recv_origin, 1)].reshape(m_per, k)
            yb = jnp.maximum(
                jnp.dot(xb, w, preferred_element_type=jnp.float32), 0.0)
            out_ref[pl.ds(recv_origin * m_per, m_per), :] = yb
            amax = jnp.maximum(amax, jnp.max(yb))

        amax_ref[pl.ds(my, 1)] = jnp.full((1, 128), amax, jnp.float32)
        for d in range(1, N_DEV):
            tgt = jnp.mod(my + d, N_DEV)
            s = pltpu.make_async_remote_copy(
                src_ref=amax_ref.at[my],
                dst_ref=amax_ref.at[my],
                send_sem=sc_send_sem,
                recv_sem=sc_recv_sem,
                device_id=(tgt,),
                device_id_type=pl.DeviceIdType.MESH,
            )
            s.start()
            s.wait_send()
        for d in range(1, N_DEV):
            src = jnp.mod(my + d, N_DEV)
            r = pltpu.make_async_remote_copy(
                src_ref=amax_ref.at[src],
                dst_ref=amax_ref.at[src],
                send_sem=sc_send_sem,
                recv_sem=sc_recv_sem,
                device_id=(my,),
                device_id_type=pl.DeviceIdType.MESH,
            )
            r.wait_recv()

        amax_g = jnp.max(amax_ref[...])
        scale = jnp.maximum(amax_g, 1e-30) / 448.0

        y = out_ref[...]
        q = (y / scale).astype(jnp.float8_e4m3fn).astype(jnp.float32)
        out_ref[...] = q * scale

    return pl.pallas_call(
        body,
        out_shape=jax.ShapeDtypeStruct((m_tot, n), jnp.float32),
        in_specs=[
            pl.BlockSpec(memory_space=pltpu.VMEM),
            pl.BlockSpec(memory_space=pltpu.VMEM),
        ],
        out_specs=pl.BlockSpec(memory_space=pltpu.VMEM),
        scratch_shapes=[
            pltpu.VMEM((N_DEV, m_per, k), jnp.bfloat16),
            pltpu.VMEM((N_DEV, 128), jnp.float32),
            pltpu.SemaphoreType.DMA,
            pltpu.SemaphoreType.DMA((N_DEV,)),
            pltpu.SemaphoreType.DMA,
            pltpu.SemaphoreType.DMA,
        ],
        compiler_params=pltpu.CompilerParams(
            collective_id=0,
            vmem_limit_bytes=100 * 1024 * 1024,
        ),
    )(x, w_mat)


# baseline (device time: 467758 ns/iter reference)
import jax
import jax.numpy as jnp
from jax import lax
from jax.experimental import pallas as pl
from jax.experimental.pallas import tpu as pltpu

N_DEV = 32


def kernel(x, w_mat):
    m_per, k = x.shape
    n = w_mat.shape[1]
    m_tot = N_DEV * m_per

    def body(x_ref, w_ref, out_ref, gx_ref, amax_ref,
             ring_send_sem, ring_recv_sems, sc_send_sem, sc_recv_sem):
        my = lax.axis_index("i")
        left = jnp.mod(my - 1, N_DEV)
        right = jnp.mod(my + 1, N_DEV)

        barrier_sem = pltpu.get_barrier_semaphore()
        pl.semaphore_signal(barrier_sem, inc=1, device_id=(left,),
                            device_id_type=pl.DeviceIdType.MESH)
        pl.semaphore_signal(barrier_sem, inc=1, device_id=(right,),
                            device_id_type=pl.DeviceIdType.MESH)
        pl.semaphore_wait(barrier_sem, 2)

        w = w_ref[...].astype(jnp.bfloat16)

        xb0 = x_ref[...].astype(jnp.bfloat16)
        gx_ref[pl.ds(my, 1)] = xb0.reshape(1, m_per, k)
        y0 = jnp.maximum(
            jnp.dot(xb0, w, preferred_element_type=jnp.float32), 0.0)
        out_ref[pl.ds(my * m_per, m_per), :] = y0
        amax = jnp.max(y0)

        for h in range(N_DEV - 1):
            send_origin = jnp.mod(my - h, N_DEV)
            recv_origin = jnp.mod(my - h - 1, N_DEV)
            send = pltpu.make_async_remote_copy(
                src_ref=gx_ref.at[send_origin],
                dst_ref=gx_ref.at[send_origin],
                send_sem=ring_send_sem,
                recv_sem=ring_recv_sems.at[send_origin],
                device_id=(right,),
                device_id_type=pl.DeviceIdType.MESH,
            )
            send.start()
            send.wait_send()
            recv = pltpu.make_async_remote_copy(
                src_ref=gx_ref.at[recv_origin],
                dst_ref=gx_ref.at[recv_origin],
                send_sem=ring_send_sem,
                recv_sem=ring_recv_sems.at[recv_origin],
                device_id=(right,),
                device_id_type=pl.DeviceIdType.MESH,
            )
            recv.wait_recv()
            xb = gx_ref[pl.ds(recv_origin, 1)].reshape(m_per, k)
            yb = jnp.maximum(
                jnp.dot(xb, w, preferred_element_type=jnp.float32), 0.0)
            out_ref[pl.ds(recv_origin * m_per, m_per), :] = yb
            amax = jnp.maximum(amax, jnp.max(yb))

        amax_ref[pl.ds(my, 1)] = jnp.full((1, 128), amax, jnp.float32)
        for d in range(1, N_DEV):
            tgt = jnp.mod(my + d, N_DEV)
            s = pltpu.make_async_remote_copy(
                src_ref=amax_ref.at[my],
                dst_ref=amax_ref.at[my],
                send_sem=sc_send_sem,
                recv_sem=sc_recv_sem,
                device_id=(tgt,),
                device_id_type=pl.DeviceIdType.MESH,
            )
            s.start()
            s.wait_send()
        for d in range(1, N_DEV):
            src = jnp.mod(my + d, N_DEV)
            r = pltpu.make_async_remote_copy(
                src_ref=amax_ref.at[src],
                dst_ref=amax_ref.at[src],
                send_sem=sc_send_sem,
                recv_sem=sc_recv_sem,
                device_id=(my,),
                device_id_type=pl.DeviceIdType.MESH,
            )
            r.wait_recv()

        amax_g = jnp.max(amax_ref[...])
        scale = jnp.maximum(amax_g, 1e-30) / 448.0

        y = out_ref[...]
        q = (y / scale).astype(jnp.float8_e4m3fn).astype(jnp.float32)
        out_ref[...] = q * scale

    return pl.pallas_call(
        body,
        out_shape=jax.ShapeDtypeStruct((m_tot, n), jnp.float32),
        in_specs=[
            pl.BlockSpec(memory_space=pltpu.VMEM),
            pl.BlockSpec(memory_space=pltpu.VMEM),
        ],
        out_specs=pl.BlockSpec(memory_space=pltpu.VMEM),
        scratch_shapes=[
            pltpu.VMEM((N_DEV, m_per, k), jnp.bfloat16),
            pltpu.VMEM((N_DEV, 128), jnp.float32),
            pltpu.SemaphoreType.DMA,
            pltpu.SemaphoreType.DMA((N_DEV,)),
            pltpu.SemaphoreType.DMA,
            pltpu.SemaphoreType.DMA,
        ],
        compiler_params=pltpu.CompilerParams(
            collective_id=0,
            vmem_limit_bytes=100 * 1024 * 1024,
        ),
    )(x, w_mat)


# device time: 230104 ns/iter; 2.0328x vs baseline; 2.0328x over previous
import jax
import jax.numpy as jnp
import numpy as np
from jax import lax
from jax.experimental import pallas as pl
from jax.experimental.pallas import tpu as pltpu

N_DEV = 32

_RING = np.array(
    [0, 1, 2, 5, 6, 7, 4, 3,
     11, 12, 15, 14, 13, 10, 9,
     17, 18, 21, 22, 23, 20, 19,
     27, 28, 31, 30, 29, 26, 25, 24,
     16, 8],
    dtype=np.int32,
)
_INV = np.argsort(_RING).astype(np.int32)

N_FWD = 16
N_BWD = 15
N_SC_SEMS = 4


def kernel(x, w_mat):
    m_per, k = x.shape
    n = w_mat.shape[1]
    m_tot = N_DEV * m_per

    def body(ring_ref, inv_ref, x_ref, w_ref, out_ref, gx_ref, amax_ref,
             ring_send_sems, ring_recv_sems, sc_send_sems, sc_recv_sem):
        my = lax.axis_index("i")
        r = inv_ref[my]
        left = ring_ref[jnp.mod(r - 1, N_DEV)]
        right = ring_ref[jnp.mod(r + 1, N_DEV)]

        barrier_sem = pltpu.get_barrier_semaphore()
        pl.semaphore_signal(barrier_sem, inc=1, device_id=(left,),
                            device_id_type=pl.DeviceIdType.MESH)
        pl.semaphore_signal(barrier_sem, inc=1, device_id=(right,),
                            device_id_type=pl.DeviceIdType.MESH)
        pl.semaphore_wait(barrier_sem, 2)

        def mk_send(origin, tgt, sem_idx):
            return pltpu.make_async_remote_copy(
                src_ref=gx_ref.at[origin],
                dst_ref=gx_ref.at[origin],
                send_sem=ring_send_sems.at[sem_idx],
                recv_sem=ring_recv_sems.at[origin],
                device_id=(tgt,),
                device_id_type=pl.DeviceIdType.MESH,
            )

        def wait_chunk(origin):
            pltpu.make_async_remote_copy(
                src_ref=gx_ref.at[origin],
                dst_ref=gx_ref.at[origin],
                send_sem=ring_send_sems.at[0],
                recv_sem=ring_recv_sems.at[origin],
                device_id=(my,),
                device_id_type=pl.DeviceIdType.MESH,
            ).wait_recv()

        xb0 = x_ref[...].astype(jnp.bfloat16)
        gx_ref[pl.ds(my, 1)] = xb0.reshape(1, m_per, k)

        fwd = mk_send(my, right, 0)
        fwd.start()
        bwd = mk_send(my, left, 1)
        bwd.start()

        w = w_ref[...].astype(jnp.bfloat16)
        y0 = jnp.maximum(
            jnp.dot(xb0, w, preferred_element_type=jnp.float32), 0.0)
        out_ref[pl.ds(my * m_per, m_per), :] = y0
        amax = jnp.max(y0)

        def gemm_chunk(origin, amax):
            xb = gx_ref[pl.ds(origin, 1)].reshape(m_per, k)
            yb = jnp.maximum(
                jnp.dot(xb, w, preferred_element_type=jnp.float32), 0.0)
            out_ref[pl.ds(origin * m_per, m_per), :] = yb
            return jnp.maximum(amax, jnp.max(yb))

        for h in range(N_FWD):
            rf = ring_ref[jnp.mod(r - h - 1, N_DEV)]
            wait_chunk(rf)
            if h < N_FWD - 1:
                fwd.wait_send()
                fwd = mk_send(rf, right, 0)
                fwd.start()
            rb = None
            if h < N_BWD:
                rb = ring_ref[jnp.mod(r + h + 1, N_DEV)]
                wait_chunk(rb)
                if h < N_BWD - 1:
                    bwd.wait_send()
                    bwd = mk_send(rb, left, 1)
                    bwd.start()
            amax = gemm_chunk(rf, amax)
            if rb is not None:
                amax = gemm_chunk(rb, amax)
        fwd.wait_send()
        bwd.wait_send()

        amax_ref[pl.ds(my, 1)] = jnp.full((1, 128), amax, jnp.float32)
        descs = []
        for d in range(1, N_DEV):
            tgt = jnp.mod(my + d, N_DEV)
            s = pltpu.make_async_remote_copy(
                src_ref=amax_ref.at[my],
                dst_ref=amax_ref.at[my],
                send_sem=sc_send_sems.at[(d - 1) % N_SC_SEMS],
                recv_sem=sc_recv_sem,
                device_id=(tgt,),
                device_id_type=pl.DeviceIdType.MESH,
            )
            if d - 1 >= N_SC_SEMS:
                descs[d - 1 - N_SC_SEMS].wait_send()
            s.start()
            descs.append(s)
        for i in range(N_DEV - 1 - N_SC_SEMS, N_DEV - 1):
            descs[i].wait_send()
        for d in range(1, N_DEV):
            src = jnp.mod(my + d, N_DEV)
            pltpu.make_async_remote_copy(
                src_ref=amax_ref.at[src],
                dst_ref=amax_ref.at[src],
                send_sem=sc_send_sems.at[0],
                recv_sem=sc_recv_sem,
                device_id=(my,),
                device_id_type=pl.DeviceIdType.MESH,
            ).wait_recv()

        amax_g = jnp.max(amax_ref[...])
        scale = jnp.maximum(amax_g, 1e-30) / 448.0

        y = out_ref[...]
        q = (y / scale).astype(jnp.float8_e4m3fn).astype(jnp.float32)
        out_ref[...] = q * scale

    ring = jnp.asarray(_RING)
    inv = jnp.asarray(_INV)

    return pl.pallas_call(
        body,
        out_shape=jax.ShapeDtypeStruct((m_tot, n), jnp.float32),
        in_specs=[
            pl.BlockSpec(memory_space=pltpu.SMEM),
            pl.BlockSpec(memory_space=pltpu.SMEM),
            pl.BlockSpec(memory_space=pltpu.VMEM),
            pl.BlockSpec(memory_space=pltpu.VMEM),
        ],
        out_specs=pl.BlockSpec(memory_space=pltpu.VMEM),
        scratch_shapes=[
            pltpu.VMEM((N_DEV, m_per, k), jnp.bfloat16),
            pltpu.VMEM((N_DEV, 128), jnp.float32),
            pltpu.SemaphoreType.DMA((2,)),
            pltpu.SemaphoreType.DMA((N_DEV,)),
            pltpu.SemaphoreType.DMA((N_SC_SEMS,)),
            pltpu.SemaphoreType.DMA,
        ],
        compiler_params=pltpu.CompilerParams(
            collective_id=0,
            vmem_limit_bytes=100 * 1024 * 1024,
        ),
    )(ring, inv, x, w_mat)


# device time: 204384 ns/iter; 2.2886x vs baseline; 1.1258x over previous
import jax
import jax.numpy as jnp
import numpy as np
from jax import lax
from jax.experimental import pallas as pl
from jax.experimental.pallas import tpu as pltpu

N_DEV = 32

_RING = np.array(
    [0, 1, 2, 5, 6, 7, 4, 3,
     11, 12, 15, 14, 13, 10, 9,
     17, 18, 21, 22, 23, 20, 19,
     27, 28, 31, 30, 29, 26, 25, 24,
     16, 8],
    dtype=np.int32,
)
_INV = np.argsort(_RING).astype(np.int32)

N_FWD = 16
N_BWD = 15
N_SC_SEMS = 4


def kernel(x, w_mat):
    m_per, k = x.shape
    n = w_mat.shape[1]
    m_tot = N_DEV * m_per

    def body(ring_ref, inv_ref, x_ref, w_ref, out_ref, gx_ref, amax_ref,
             ring_send_sems, ring_recv_sems, sc_send_sems, sc_recv_sem):
        my = lax.axis_index("i")
        r = inv_ref[my]
        left = ring_ref[jnp.mod(r - 1, N_DEV)]
        right = ring_ref[jnp.mod(r + 1, N_DEV)]

        barrier_sem = pltpu.get_barrier_semaphore()
        pl.semaphore_signal(barrier_sem, inc=1, device_id=(left,),
                            device_id_type=pl.DeviceIdType.MESH)
        pl.semaphore_signal(barrier_sem, inc=1, device_id=(right,),
                            device_id_type=pl.DeviceIdType.MESH)
        pl.semaphore_wait(barrier_sem, 2)

        hm = m_per // 2

        def mk_send(origin, half, tgt, dir_idx):
            return pltpu.make_async_remote_copy(
                src_ref=gx_ref.at[origin, pl.ds(half * hm, hm)],
                dst_ref=gx_ref.at[origin, pl.ds(half * hm, hm)],
                send_sem=ring_send_sems.at[dir_idx, half],
                recv_sem=ring_recv_sems.at[origin, half],
                device_id=(tgt,),
                device_id_type=pl.DeviceIdType.MESH,
            )

        def wait_half(origin, half):
            pltpu.make_async_remote_copy(
                src_ref=gx_ref.at[origin, pl.ds(half * hm, hm)],
                dst_ref=gx_ref.at[origin, pl.ds(half * hm, hm)],
                send_sem=ring_send_sems.at[0, half],
                recv_sem=ring_recv_sems.at[origin, half],
                device_id=(my,),
                device_id_type=pl.DeviceIdType.MESH,
            ).wait_recv()

        xb0 = x_ref[...].astype(jnp.bfloat16)
        gx_ref[pl.ds(my, 1)] = xb0.reshape(1, m_per, k)

        fwd = [mk_send(my, 0, right, 0), mk_send(my, 1, right, 0)]
        bwd = [mk_send(my, 0, left, 1), mk_send(my, 1, left, 1)]
        for d in (*fwd, *bwd):
            d.start()

        w = w_ref[...].astype(jnp.bfloat16)
        y0 = jnp.maximum(
            jnp.dot(xb0, w, preferred_element_type=jnp.float32), 0.0)
        out_ref[pl.ds(my * m_per, m_per), :] = y0
        amax = jnp.max(y0)

        def gemm_chunk(origin, amax):
            xb = gx_ref[pl.ds(origin, 1)].reshape(m_per, k)
            yb = jnp.maximum(
                jnp.dot(xb, w, preferred_element_type=jnp.float32), 0.0)
            out_ref[pl.ds(origin * m_per, m_per), :] = yb
            return jnp.maximum(amax, jnp.max(yb))

        for h in range(N_FWD):
            rf = ring_ref[jnp.mod(r - h - 1, N_DEV)]
            rb = None
            if h < N_BWD:
                rb = ring_ref[jnp.mod(r + h + 1, N_DEV)]
            for half in (0, 1):
                wait_half(rf, half)
                if h < N_FWD - 1:
                    fwd[half].wait_send()
                    fwd[half] = mk_send(rf, half, right, 0)
                    fwd[half].start()
                if rb is not None:
                    wait_half(rb, half)
                    if h < N_BWD - 1:
                        bwd[half].wait_send()
                        bwd[half] = mk_send(rb, half, left, 1)
                        bwd[half].start()
            amax = gemm_chunk(rf, amax)
            if rb is not None:
                amax = gemm_chunk(rb, amax)
        for d in (*fwd, *bwd):
            d.wait_send()

        amax_ref[pl.ds(my, 1)] = jnp.full((1, 128), amax, jnp.float32)
        descs = []
        for d in range(1, N_DEV):
            tgt = jnp.mod(my + d, N_DEV)
            s = pltpu.make_async_remote_copy(
                src_ref=amax_ref.at[my],
                dst_ref=amax_ref.at[my],
                send_sem=sc_send_sems.at[(d - 1) % N_SC_SEMS],
                recv_sem=sc_recv_sem,
                device_id=(tgt,),
                device_id_type=pl.DeviceIdType.MESH,
            )
            if d - 1 >= N_SC_SEMS:
                descs[d - 1 - N_SC_SEMS].wait_send()
            s.start()
            descs.append(s)
        for i in range(N_DEV - 1 - N_SC_SEMS, N_DEV - 1):
            descs[i].wait_send()
        for d in range(1, N_DEV):
            src = jnp.mod(my + d, N_DEV)
            pltpu.make_async_remote_copy(
                src_ref=amax_ref.at[src],
                dst_ref=amax_ref.at[src],
                send_sem=sc_send_sems.at[0],
                recv_sem=sc_recv_sem,
                device_id=(my,),
                device_id_type=pl.DeviceIdType.MESH,
            ).wait_recv()

        amax_g = jnp.max(amax_ref[...])
        scale = jnp.maximum(amax_g, 1e-30) / 448.0

        y = out_ref[...]
        q = (y / scale).astype(jnp.float8_e4m3fn).astype(jnp.float32)
        out_ref[...] = q * scale

    ring = jnp.asarray(_RING)
    inv = jnp.asarray(_INV)

    return pl.pallas_call(
        body,
        out_shape=jax.ShapeDtypeStruct((m_tot, n), jnp.float32),
        in_specs=[
            pl.BlockSpec(memory_space=pltpu.SMEM),
            pl.BlockSpec(memory_space=pltpu.SMEM),
            pl.BlockSpec(memory_space=pltpu.VMEM),
            pl.BlockSpec(memory_space=pltpu.VMEM),
        ],
        out_specs=pl.BlockSpec(memory_space=pltpu.VMEM),
        scratch_shapes=[
            pltpu.VMEM((N_DEV, m_per, k), jnp.bfloat16),
            pltpu.VMEM((N_DEV, 128), jnp.float32),
            pltpu.SemaphoreType.DMA((2, 2)),
            pltpu.SemaphoreType.DMA((N_DEV, 2)),
            pltpu.SemaphoreType.DMA((N_SC_SEMS,)),
            pltpu.SemaphoreType.DMA,
        ],
        compiler_params=pltpu.CompilerParams(
            collective_id=0,
            vmem_limit_bytes=100 * 1024 * 1024,
        ),
    )(ring, inv, x, w_mat)
